# Initial kernel scaffold; baseline (speedup 1.0000x reference)
import jax
import jax.numpy as jnp
from jax.experimental import pallas as pl


def kernel(x, edge_index, edge_attr, bn_gamma, bn_beta, in_W1, in_b1, in_W2, in_b2, conv_W1, conv_b1, conv_W2, conv_b2, eg_W1, eg_b1, eg_W2, eg_b2):
    raise NotImplementedError("write your pallas kernel here")


# speedup vs baseline: 1.8048x; 1.8048x over previous
import functools

import jax
import jax.numpy as jnp
from jax import lax
from jax.experimental import pallas as pl
from jax.experimental.pallas import tpu as pltpu
from jax.experimental.pallas import tpu_sc as plsc

NN = 100000
EDGES = 1600000
HID = 32
CH = 64
EP = 1605632
ROWS128 = EP // 128
NW = 32
PER_W = EP // NW
CHUNK = 1024
G_ITERS = PER_W // CHUNK
PER_TEC_S = EP // 16
S_ITERS = PER_TEC_S // CHUNK
QUART = NN // 4
SPQ_ROWS = 25088
ZROWS = SPQ_ROWS // 16
BN_EPS = 1e-5

def _mesh():
    return plsc.VectorSubcoreMesh(core_axis_name="c", subcore_axis_name="s")


def _stats_body(x_ref, s_ref, ss_ref):
    i = pl.program_id(0)

    @pl.when(i == 0)
    def _():
        s_ref[...] = jnp.zeros_like(s_ref)
        ss_ref[...] = jnp.zeros_like(ss_ref)

    xb = x_ref[...]
    s_ref[...] += jnp.sum(xb, axis=0, keepdims=True)
    ss_ref[...] += jnp.sum(xb * xb, axis=0, keepdims=True)


def _node_stats(x, bn):
    return pl.pallas_call(
        _stats_body,
        grid=(50,),
        in_specs=[pl.BlockSpec((bn, 3), lambda i: (i, 0))],
        out_specs=[pl.BlockSpec((1, 3), lambda i: (0, 0)),
                   pl.BlockSpec((1, 3), lambda i: (0, 0))],
        out_shape=[jax.ShapeDtypeStruct((1, 3), jnp.float32),
                   jax.ShapeDtypeStruct((1, 3), jnp.float32)],
    )(x)


def _node1_body(x_ref, s_ref, ss_ref, g_ref, b_ref, w1_ref, b1_ref,
                w2_ref, b2_ref, pw_ref, qw_ref, cb1_ref,
                p_ref, q_ref, xo_ref):
    mean = s_ref[...] / NN
    var = ss_ref[...] / NN - mean * mean
    scale = g_ref[...] * lax.rsqrt(var + BN_EPS)
    shift = b_ref[...] - mean * scale
    X = x_ref[...] * scale + shift
    dot = functools.partial(jnp.dot, preferred_element_type=jnp.float32)
    T = jnp.maximum(dot(X, w1_ref[...]) + b1_ref[...], 0.0)
    H = jnp.tanh(dot(T, w2_ref[...]) + b2_ref[...])
    pw = pw_ref[...]
    qw = qw_ref[...]
    p_ref[...] = dot(H, pw[:HID]) + dot(X, pw[HID:]) + cb1_ref[...]
    q_ref[...] = dot(H, qw[:HID]) + dot(X, qw[HID:])
    xo_ref[...] = X


def _node_stage1(x, s, ss, gamma, beta, w1, b1, w2, b2, pw, qw, cb1, bn):
    full2 = lambda a: pl.BlockSpec(a.shape, lambda i: (0, 0))
    return pl.pallas_call(
        _node1_body,
        grid=(NN // bn,),
        in_specs=[pl.BlockSpec((bn, 3), lambda i: (i, 0)),
                  full2(s), full2(ss), full2(gamma), full2(beta),
                  full2(w1), full2(b1), full2(w2), full2(b2),
                  full2(pw), full2(qw), full2(cb1)],
        out_specs=[pl.BlockSpec((bn, CH), lambda i: (i, 0)),
                   pl.BlockSpec((bn, CH), lambda i: (i, 0)),
                   pl.BlockSpec((bn, 3), lambda i: (i, 0))],
        out_shape=[jax.ShapeDtypeStruct((NN, CH), jnp.float32),
                   jax.ShapeDtypeStruct((NN, CH), jnp.float32),
                   jax.ShapeDtypeStruct((NN, 3), jnp.float32)],
    )(x, s, ss, gamma, beta, w1, b1, w2, b2, pw, qw, cb1)


def _node2_body(h2_ref, x_ref, aw_ref, bw_ref, eb1_ref, u_ref, v_ref):
    dot = functools.partial(jnp.dot, preferred_element_type=jnp.float32)
    aw = aw_ref[...]
    bw = bw_ref[...]
    h2 = h2_ref[...]
    X = x_ref[...]
    u_ref[...] = dot(h2, aw[:HID]) + dot(X, aw[HID:])
    v_ref[...] = dot(h2, bw[:HID]) + dot(X, bw[HID:]) + eb1_ref[...]


def _node_stage2(h2, X, aw, bw, eb1, bn):
    full2 = lambda a: pl.BlockSpec(a.shape, lambda i: (0, 0))
    return pl.pallas_call(
        _node2_body,
        grid=(NN // bn,),
        in_specs=[pl.BlockSpec((bn, HID), lambda i: (i, 0)),
                  pl.BlockSpec((bn, 3), lambda i: (i, 0)),
                  full2(aw), full2(bw), full2(eb1)],
        out_specs=[pl.BlockSpec((bn, CH), lambda i: (i, 0)),
                   pl.BlockSpec((bn, CH), lambda i: (i, 0))],
        out_shape=[jax.ShapeDtypeStruct((NN, CH), jnp.float32),
                   jax.ShapeDtypeStruct((NN, CH), jnp.float32)],
    )(h2, X, aw, bw, eb1)


def _gather_sum(tabA, tabB, iA2d, iB2d):
    @functools.partial(
        pl.kernel,
        out_type=jax.ShapeDtypeStruct((EP, CH), jnp.float32),
        mesh=_mesh(),
        scratch_types=[pltpu.VMEM((8, 128), jnp.int32),
                       pltpu.VMEM((8, 128), jnp.int32),
                       pltpu.VMEM((128, CH), jnp.float32),
                       pltpu.VMEM((128, CH), jnp.float32),
                       pltpu.VMEM((CHUNK, CH), jnp.float32),
                       pltpu.SemaphoreType.DMA],
        compiler_params=pltpu.CompilerParams(use_tc_tiling_on_sc=False),
    )
    def k(tA, tB, iA, iB, out, ia_v, ib_v, abuf, bbuf, obuf, sem):
        wid = lax.axis_index("s") * 2 + lax.axis_index("c")
        row0 = wid * (PER_W // 128)

        def chunk(i):
            pltpu.sync_copy(iA.at[pl.ds(row0 + i * 8, 8)], ia_v)
            pltpu.sync_copy(iB.at[pl.ds(row0 + i * 8, 8)], ib_v)
            for j in range(8):
                cpa = pltpu.async_copy(tA.at[ia_v.at[j]], abuf, sem)
                cpb = pltpu.async_copy(tB.at[ib_v.at[j]], bbuf, sem)
                cpa.wait()
                cpb.wait()

                def add_row(r, j=j):
                    for c4 in range(CH // 16):
                        sl = pl.ds(c4 * 16, 16)
                        obuf[j * 128 + r, sl] = abuf[r, sl] + bbuf[r, sl]

                pl.loop(0, 128)(add_row)
            pltpu.sync_copy(
                obuf, out.at[pl.ds(wid * PER_W + i * CHUNK, CHUNK)])

        pl.loop(0, G_ITERS)(chunk)

    return k(tabA, tabB, iA2d, iB2d)


def _scatter_h2(msg, dsts2d, zeros):
    @functools.partial(
        pl.kernel,
        out_type=jax.ShapeDtypeStruct((NN, HID), jnp.float32),
        mesh=_mesh(),
        scratch_types=[pltpu.VMEM((8, 128), jnp.int32),
                       pltpu.VMEM((8, 128), jnp.int32),
                       pltpu.VMEM((CHUNK, HID), jnp.float32),
                       pltpu.VMEM_SHARED((SPQ_ROWS, HID), jnp.float32),
                       pltpu.SemaphoreType.DMA],
        compiler_params=pltpu.CompilerParams(use_tc_tiling_on_sc=False),
    )
    def k(msg_hbm, dsts_hbm, zeros_hbm, out, idx_v, lidx_v, mbuf, acc, sem):
        c = lax.axis_index("c")
        t = lax.axis_index("s")
        for p in range(2):
            base = (2 * c + p) * QUART
            pltpu.sync_copy(zeros_hbm, acc.at[pl.ds(t * ZROWS, ZROWS)])
            plsc.subcore_barrier()

            def chunk(i, base=base):
                pltpu.sync_copy(
                    dsts_hbm.at[pl.ds(t * (PER_TEC_S // 128) + i * 8, 8)],
                    idx_v)
                pltpu.sync_copy(
                    msg_hbm.at[pl.ds(t * PER_TEC_S + i * CHUNK, CHUNK)],
                    mbuf)
                for j in range(8):
                    def to_local(kk, j=j, base=base):
                        sl = pl.ds(kk * 16, 16)
                        v = idx_v[j, sl] - base
                        m = (v >= 0) & (v < QUART)
                        lidx_v[j, sl] = jnp.where(m, v, QUART)

                    pl.loop(0, 8)(to_local)
                for j in range(8):
                    pltpu.sync_copy(mbuf.at[pl.ds(j * 128, 128)],
                                    acc.at[lidx_v.at[j]], add=True)

            pl.loop(0, S_ITERS)(chunk)
            plsc.subcore_barrier()

            @pl.when(t == 0)
            def _(base=base):
                pltpu.sync_copy(acc.at[pl.ds(0, QUART)],
                                out.at[pl.ds(base, QUART)])

            plsc.subcore_barrier()

    return k(msg, dsts2d, zeros)


def _b2_body(g_ref, ea_ref, c1_ref, w2_ref, b2_ref, o_ref):
    dot = functools.partial(jnp.dot, preferred_element_type=jnp.float32)
    h = jnp.maximum(g_ref[...] + dot(ea_ref[...], c1_ref[...]), 0.0)
    o_ref[...] = jnp.tanh(dot(h, w2_ref[...]) + b2_ref[...])


def _edge_msg(G, ea, c1, w2, b2, be):
    full2 = lambda a: pl.BlockSpec(a.shape, lambda i: (0, 0))
    return pl.pallas_call(
        _b2_body,
        grid=(EP // be,),
        in_specs=[pl.BlockSpec((be, CH), lambda i: (i, 0)),
                  pl.BlockSpec((be, 4), lambda i: (i, 0)),
                  full2(c1), full2(w2), full2(b2)],
        out_specs=[pl.BlockSpec((be, HID), lambda i: (i, 0))],
        out_shape=[jax.ShapeDtypeStruct((EP, HID), jnp.float32)],
    )(G, ea, c1, w2, b2)[0]


def _b3_body(g_ref, ea_ref, c2_ref, w_ref, b_ref, o_ref):
    dot = functools.partial(jnp.dot, preferred_element_type=jnp.float32)
    h = jnp.maximum(g_ref[...] + dot(ea_ref[...], c2_ref[...]), 0.0)
    o_ref[...] = jax.nn.sigmoid(dot(h, w_ref[...]) + b_ref[...])


def _edge_score(G2, ea, c2, w, b, be):
    full2 = lambda a: pl.BlockSpec(a.shape, lambda i: (0, 0))
    return pl.pallas_call(
        _b3_body,
        grid=(EP // be,),
        in_specs=[pl.BlockSpec((be, CH), lambda i: (i, 0)),
                  pl.BlockSpec((be, 4), lambda i: (i, 0)),
                  full2(c2), full2(w), full2(b)],
        out_specs=[pl.BlockSpec((be, 1), lambda i: (i, 0))],
        out_shape=[jax.ShapeDtypeStruct((EP, 1), jnp.float32)],
    )(G2, ea, c2, w, b)[0]


def kernel(x, edge_index, edge_attr, bn_gamma, bn_beta,
           in_W1, in_b1, in_W2, in_b2,
           conv_W1, conv_b1, conv_W2, conv_b2,
           eg_W1, eg_b1, eg_W2, eg_b2):
    src = edge_index[0]
    dst = edge_index[1]
    pad = EP - EDGES
    src_g = jnp.pad(src, (0, pad)).reshape(ROWS128, 128)
    dst_g = jnp.pad(dst, (0, pad)).reshape(ROWS128, 128)
    dst_s = jnp.pad(dst, (0, pad), constant_values=NN).reshape(ROWS128, 128)
    ea_p = jnp.pad(edge_attr, ((0, pad), (0, 0)))

    A1 = conv_W1[:HID + 3]
    B1 = conv_W1[HID + 3:2 * (HID + 3)]
    C1 = conv_W1[2 * (HID + 3):]
    pw = A1 - B1
    qw = B1
    A2 = eg_W1[:HID + 3]
    B2w = eg_W1[HID + 3:2 * (HID + 3)]
    C2 = eg_W1[2 * (HID + 3):]

    r1 = lambda a: a.reshape(1, -1)
    s, ss = _node_stats(x, 2000)
    P, Q, X = _node_stage1(x, s, ss, r1(bn_gamma), r1(bn_beta),
                           in_W1, r1(in_b1), in_W2, r1(in_b2),
                           pw, qw, r1(conv_b1), 2000)

    G1 = _gather_sum(P, Q, dst_g, src_g)
    msg = _edge_msg(G1, ea_p, C1, conv_W2, r1(conv_b2), 2048)
    zeros = jnp.zeros((ZROWS, HID), jnp.float32)
    H2 = _scatter_h2(msg, dst_s, zeros)

    U, V = _node_stage2(H2, X, A2, B2w, r1(eg_b1), 2000)
    G2 = _gather_sum(U, V, src_g, dst_g)
    outp = _edge_score(G2, ea_p, C2, eg_W2, r1(eg_b2), 2048)
    out = outp[:EDGES, 0]
    return (out, jnp.zeros((NN, 3), jnp.float32))


# speedup vs baseline: 2.1157x; 1.1723x over previous
import functools

import jax
import jax.numpy as jnp
from jax import lax
from jax.experimental import pallas as pl
from jax.experimental.pallas import tpu as pltpu
from jax.experimental.pallas import tpu_sc as plsc

NN = 100000
EDGES = 1600000
HID = 32
CH = 64
EP = 1605632
ROWS128 = EP // 128
NW = 32
PER_W = EP // NW
CHUNK = 1024
G_ITERS = PER_W // CHUNK
PER_TEC_S = EP // 16
S_ITERS = PER_TEC_S // CHUNK
QUART = NN // 4
SPQ_ROWS = 25088
ZROWS = SPQ_ROWS // 16
BN_EPS = 1e-5


def _mesh():
    return plsc.VectorSubcoreMesh(core_axis_name="c", subcore_axis_name="s")


def _stats_body(x_ref, s_ref, ss_ref):
    i = pl.program_id(0)

    @pl.when(i == 0)
    def _():
        s_ref[...] = jnp.zeros_like(s_ref)
        ss_ref[...] = jnp.zeros_like(ss_ref)

    xb = x_ref[...]
    s_ref[...] += jnp.sum(xb, axis=0, keepdims=True)
    ss_ref[...] += jnp.sum(xb * xb, axis=0, keepdims=True)


def _node_stats(x, bn):
    return pl.pallas_call(
        _stats_body,
        grid=(NN // bn,),
        in_specs=[pl.BlockSpec((bn, 3), lambda i: (i, 0))],
        out_specs=[pl.BlockSpec((1, 3), lambda i: (0, 0)),
                   pl.BlockSpec((1, 3), lambda i: (0, 0))],
        out_shape=[jax.ShapeDtypeStruct((1, 3), jnp.float32),
                   jax.ShapeDtypeStruct((1, 3), jnp.float32)],
    )(x)


def _bn_scale_shift(s_ref, ss_ref, g_ref, b_ref):
    mean = s_ref[...] / NN
    var = ss_ref[...] / NN - mean * mean
    scale = g_ref[...] * lax.rsqrt(var + BN_EPS)
    shift = b_ref[...] - mean * scale
    return scale, shift


def _node1_body(xe_ref, xo_ref, s_ref, ss_ref, g_ref, b_ref, w1_ref, b1_ref,
                w2_ref, b2_ref, pw_ref, qw_ref, cb1_ref, p_ref, q_ref):
    scale, shift = _bn_scale_shift(s_ref, ss_ref, g_ref, b_ref)
    dot = functools.partial(jnp.dot, preferred_element_type=jnp.float32)
    pw = pw_ref[...]
    qw = qw_ref[...]

    def half(x_ref):
        X = x_ref[...] * scale + shift
        T = jnp.maximum(dot(X, w1_ref[...]) + b1_ref[...], 0.0)
        H = jnp.tanh(dot(T, w2_ref[...]) + b2_ref[...])
        P = dot(H, pw[:HID]) + dot(X, pw[HID:]) + cb1_ref[...]
        Q = dot(H, qw[:HID]) + dot(X, qw[HID:])
        return P, Q

    Pe, Qe = half(xe_ref)
    Po, Qo = half(xo_ref)
    p_ref[...] = jnp.concatenate([Pe, Po], axis=1)
    q_ref[...] = jnp.concatenate([Qe, Qo], axis=1)


def _node_stage1(xe, xo, s, ss, gamma, beta, w1, b1, w2, b2, pw, qw, cb1, bn2):
    full2 = lambda a: pl.BlockSpec(a.shape, lambda i: (0, 0))
    return pl.pallas_call(
        _node1_body,
        grid=(NN // 2 // bn2,),
        in_specs=[pl.BlockSpec((bn2, 3), lambda i: (i, 0)),
                  pl.BlockSpec((bn2, 3), lambda i: (i, 0)),
                  full2(s), full2(ss), full2(gamma), full2(beta),
                  full2(w1), full2(b1), full2(w2), full2(b2),
                  full2(pw), full2(qw), full2(cb1)],
        out_specs=[pl.BlockSpec((bn2, 2 * CH), lambda i: (i, 0)),
                   pl.BlockSpec((bn2, 2 * CH), lambda i: (i, 0))],
        out_shape=[jax.ShapeDtypeStruct((NN // 2, 2 * CH), jnp.float32),
                   jax.ShapeDtypeStruct((NN // 2, 2 * CH), jnp.float32)],
    )(xe, xo, s, ss, gamma, beta, w1, b1, w2, b2, pw, qw, cb1)


def _node2_body(h2e_ref, h2o_ref, xe_ref, xo_ref, s_ref, ss_ref, g_ref, b_ref,
                aw_ref, bw_ref, eb1_ref, u_ref, v_ref):
    scale, shift = _bn_scale_shift(s_ref, ss_ref, g_ref, b_ref)
    dot = functools.partial(jnp.dot, preferred_element_type=jnp.float32)
    aw = aw_ref[...]
    bw = bw_ref[...]

    def half(h2_ref, x_ref):
        X = x_ref[...] * scale + shift
        h2 = h2_ref[...]
        U = dot(h2, aw[:HID]) + dot(X, aw[HID:])
        V = dot(h2, bw[:HID]) + dot(X, bw[HID:]) + eb1_ref[...]
        return U, V

    Ue, Ve = half(h2e_ref, xe_ref)
    Uo, Vo = half(h2o_ref, xo_ref)
    u_ref[...] = jnp.concatenate([Ue, Uo], axis=1)
    v_ref[...] = jnp.concatenate([Ve, Vo], axis=1)


def _node_stage2(h2e, h2o, xe, xo, s, ss, gamma, beta, aw, bw, eb1, bn2):
    full2 = lambda a: pl.BlockSpec(a.shape, lambda i: (0, 0))
    return pl.pallas_call(
        _node2_body,
        grid=(NN // 2 // bn2,),
        in_specs=[pl.BlockSpec((bn2, HID), lambda i: (i, 0)),
                  pl.BlockSpec((bn2, HID), lambda i: (i, 0)),
                  pl.BlockSpec((bn2, 3), lambda i: (i, 0)),
                  pl.BlockSpec((bn2, 3), lambda i: (i, 0)),
                  full2(s), full2(ss), full2(gamma), full2(beta),
                  full2(aw), full2(bw), full2(eb1)],
        out_specs=[pl.BlockSpec((bn2, 2 * CH), lambda i: (i, 0)),
                   pl.BlockSpec((bn2, 2 * CH), lambda i: (i, 0))],
        out_shape=[jax.ShapeDtypeStruct((NN // 2, 2 * CH), jnp.float32),
                   jax.ShapeDtypeStruct((NN // 2, 2 * CH), jnp.float32)],
    )(h2e, h2o, xe, xo, s, ss, gamma, beta, aw, bw, eb1)


def _gather_sum(tabA, tabB, iA2d, iB2d):
    @functools.partial(
        pl.kernel,
        out_type=[jax.ShapeDtypeStruct((EP // 4, 2 * CH), jnp.float32),
                  jax.ShapeDtypeStruct((EP // 4, 2 * CH), jnp.float32)],
        mesh=_mesh(),
        scratch_types=[pltpu.VMEM((8, 128), jnp.int32),
                       pltpu.VMEM((8, 128), jnp.int32),
                       pltpu.VMEM((2, 128, CH), jnp.float32),
                       pltpu.VMEM((2, 128, CH), jnp.float32),
                       pltpu.VMEM((CHUNK // 4, 2 * CH), jnp.float32),
                       pltpu.VMEM((CHUNK // 4, 2 * CH), jnp.float32),
                       pltpu.SemaphoreType.DMA((2,))],
        compiler_params=pltpu.CompilerParams(use_tc_tiling_on_sc=False),
    )
    def k(tA, tB, iA, iB, outA, outB, ia_v, ib_v, abuf, bbuf, oa, ob, sem):
        wid = lax.axis_index("s") * 2 + lax.axis_index("c")
        row0 = wid * (PER_W // 128)

        def chunk(i):
            pltpu.sync_copy(iA.at[pl.ds(row0 + i * 8, 8)], ia_v)
            pltpu.sync_copy(iB.at[pl.ds(row0 + i * 8, 8)], ib_v)
            cps = [pltpu.async_copy(tA.at[ia_v.at[0]], abuf.at[0], sem.at[0]),
                   pltpu.async_copy(tB.at[ib_v.at[0]], bbuf.at[0], sem.at[0])]
            for j in range(8):
                b = j % 2
                nb = (j + 1) % 2
                if j < 7:
                    cps += [pltpu.async_copy(tA.at[ia_v.at[j + 1]],
                                             abuf.at[nb], sem.at[nb]),
                            pltpu.async_copy(tB.at[ib_v.at[j + 1]],
                                             bbuf.at[nb], sem.at[nb])]
                cps.pop(0).wait()
                cps.pop(0).wait()

                def quad(r4, j=j, b=b):
                    row = j * 32 + r4
                    for t in range(4):
                        dst = oa if t < 2 else ob
                        off = (t % 2) * CH
                        for c4 in range(CH // 16):
                            sl = pl.ds(c4 * 16, 16)
                            dst[row, pl.ds(off + c4 * 16, 16)] = (
                                abuf[b, 4 * r4 + t, sl]
                                + bbuf[b, 4 * r4 + t, sl])

                pl.loop(0, 32)(quad)
            base = wid * (PER_W // 4) + i * (CHUNK // 4)
            pltpu.sync_copy(oa, outA.at[pl.ds(base, CHUNK // 4)])
            pltpu.sync_copy(ob, outB.at[pl.ds(base, CHUNK // 4)])

        pl.loop(0, G_ITERS)(chunk)

    return k(tabA, tabB, iA2d, iB2d)


def _scatter_h2(msg, dsts2d, zeros):
    @functools.partial(
        pl.kernel,
        out_type=jax.ShapeDtypeStruct((NN, HID), jnp.float32),
        mesh=_mesh(),
        scratch_types=[pltpu.VMEM((8, 128), jnp.int32),
                       pltpu.VMEM((8, 128), jnp.int32),
                       pltpu.VMEM((CHUNK, HID), jnp.float32),
                       pltpu.VMEM_SHARED((SPQ_ROWS, HID), jnp.float32),
                       pltpu.SemaphoreType.DMA],
        compiler_params=pltpu.CompilerParams(use_tc_tiling_on_sc=False),
    )
    def k(msg_hbm, dsts_hbm, zeros_hbm, out, idx_v, lidx_v, mbuf, acc, sem):
        c = lax.axis_index("c")
        t = lax.axis_index("s")
        for p in range(2):
            base = (2 * c + p) * QUART
            pltpu.sync_copy(zeros_hbm, acc.at[pl.ds(t * ZROWS, ZROWS)])
            plsc.subcore_barrier()

            def chunk(i, base=base):
                pltpu.sync_copy(
                    dsts_hbm.at[pl.ds(t * (PER_TEC_S // 128) + i * 8, 8)],
                    idx_v)
                pltpu.sync_copy(
                    msg_hbm.at[pl.ds(t * PER_TEC_S + i * CHUNK, CHUNK)],
                    mbuf)
                for j in range(8):
                    def to_local(kk, j=j, base=base):
                        sl = pl.ds(kk * 16, 16)
                        v = idx_v[j, sl] - base
                        m = (v >= 0) & (v < QUART)
                        lidx_v[j, sl] = jnp.where(m, v, QUART)

                    pl.loop(0, 8)(to_local)
                cps = [pltpu.async_copy(mbuf.at[pl.ds(j * 128, 128)],
                                        acc.at[lidx_v.at[j]], sem, add=True)
                       for j in range(8)]
                for cp in cps:
                    cp.wait()

            pl.loop(0, S_ITERS)(chunk)
            plsc.subcore_barrier()

            @pl.when(t == 0)
            def _(base=base):
                pltpu.sync_copy(acc.at[pl.ds(0, QUART)],
                                out.at[pl.ds(base, QUART)])

            plsc.subcore_barrier()

    return k(msg, dsts2d, zeros)


def _b2_body(ga_ref, gb_ref, eaa_ref, eab_ref, c1d_ref, w2d_ref, b2d_ref,
             o_ref):
    dot = functools.partial(jnp.dot, preferred_element_type=jnp.float32)
    c1d = c1d_ref[...]
    w2d = w2d_ref[...]
    b2d = b2d_ref[...]
    ha = jnp.maximum(ga_ref[...] + dot(eaa_ref[...], c1d), 0.0)
    hb = jnp.maximum(gb_ref[...] + dot(eab_ref[...], c1d), 0.0)
    msga = jnp.tanh(dot(ha, w2d) + b2d)
    msgb = jnp.tanh(dot(hb, w2d) + b2d)
    o_ref[...] = jnp.concatenate([msga, msgb], axis=1)


def _edge_msg(GA, GB, ea_a, ea_b, c1d, w2d, b2d, be):
    full2 = lambda a: pl.BlockSpec(a.shape, lambda i: (0, 0))
    be4 = be // 4
    return pl.pallas_call(
        _b2_body,
        grid=(EP // be,),
        in_specs=[pl.BlockSpec((be4, 2 * CH), lambda i: (i, 0)),
                  pl.BlockSpec((be4, 2 * CH), lambda i: (i, 0)),
                  pl.BlockSpec((be4, 8), lambda i: (i, 0)),
                  pl.BlockSpec((be4, 8), lambda i: (i, 0)),
                  full2(c1d), full2(w2d), full2(b2d)],
        out_specs=[pl.BlockSpec((be4, 4 * HID), lambda i: (i, 0))],
        out_shape=[jax.ShapeDtypeStruct((EP // 4, 4 * HID), jnp.float32)],
    )(GA, GB, ea_a, ea_b, c1d, w2d, b2d)[0]


def _b3_body(ga_ref, gb_ref, eaa_ref, eab_ref, c2d_ref, wd_ref, bd_ref,
             o_ref):
    dot = functools.partial(jnp.dot, preferred_element_type=jnp.float32)
    c2d = c2d_ref[...]
    wd = wd_ref[...]
    bd = bd_ref[...]
    ha = jnp.maximum(ga_ref[...] + dot(eaa_ref[...], c2d), 0.0)
    hb = jnp.maximum(gb_ref[...] + dot(eab_ref[...], c2d), 0.0)
    sa = jax.nn.sigmoid(dot(ha, wd) + bd)
    sb = jax.nn.sigmoid(dot(hb, wd) + bd)
    o_ref[...] = jnp.concatenate([sa, sb], axis=1)


def _edge_score(GA, GB, ea_a, ea_b, c2d, wd, bd, be):
    full2 = lambda a: pl.BlockSpec(a.shape, lambda i: (0, 0))
    be4 = be // 4
    return pl.pallas_call(
        _b3_body,
        grid=(EP // be,),
        in_specs=[pl.BlockSpec((be4, 2 * CH), lambda i: (i, 0)),
                  pl.BlockSpec((be4, 2 * CH), lambda i: (i, 0)),
                  pl.BlockSpec((be4, 8), lambda i: (i, 0)),
                  pl.BlockSpec((be4, 8), lambda i: (i, 0)),
                  full2(c2d), full2(wd), full2(bd)],
        out_specs=[pl.BlockSpec((be4, 4), lambda i: (i, 0))],
        out_shape=[jax.ShapeDtypeStruct((EP // 4, 4), jnp.float32)],
    )(GA, GB, ea_a, ea_b, c2d, wd, bd)[0]


def _blockdiag2(w):
    z = jnp.zeros_like(w)
    return jnp.concatenate(
        [jnp.concatenate([w, z], axis=1), jnp.concatenate([z, w], axis=1)],
        axis=0)


def kernel(x, edge_index, edge_attr, bn_gamma, bn_beta,
           in_W1, in_b1, in_W2, in_b2,
           conv_W1, conv_b1, conv_W2, conv_b2,
           eg_W1, eg_b1, eg_W2, eg_b2):
    src = edge_index[0]
    dst = edge_index[1]
    pad = EP - EDGES
    src_g = jnp.pad(src, (0, pad)).reshape(ROWS128, 128)
    dst_g = jnp.pad(dst, (0, pad)).reshape(ROWS128, 128)
    dst_s = jnp.pad(dst, (0, pad), constant_values=NN).reshape(ROWS128, 128)
    ea4 = jnp.pad(edge_attr, ((0, pad), (0, 0))).reshape(EP // 4, 16)
    ea_a = ea4[:, :8]
    ea_b = ea4[:, 8:]

    A1 = conv_W1[:HID + 3]
    B1 = conv_W1[HID + 3:2 * (HID + 3)]
    C1 = conv_W1[2 * (HID + 3):]
    pw = A1 - B1
    qw = B1
    A2 = eg_W1[:HID + 3]
    B2w = eg_W1[HID + 3:2 * (HID + 3)]
    C2 = eg_W1[2 * (HID + 3):]

    c1d = _blockdiag2(C1)
    c2d = _blockdiag2(C2)
    w2d = _blockdiag2(conv_W2)
    b2d = jnp.tile(conv_b2, 2).reshape(1, 2 * HID)
    wd = _blockdiag2(eg_W2)
    bd = jnp.tile(eg_b2, 2).reshape(1, 2)

    r1 = lambda a: a.reshape(1, -1)
    xe = x[0::2]
    xo = x[1::2]
    s, ss = _node_stats(x, 2000)
    P2, Q2 = _node_stage1(xe, xo, s, ss, r1(bn_gamma), r1(bn_beta),
                          in_W1, r1(in_b1), in_W2, r1(in_b2),
                          pw, qw, r1(conv_b1), 1000)

    GA1, GB1 = _gather_sum(P2.reshape(NN, CH), Q2.reshape(NN, CH),
                           dst_g, src_g)
    msg = _edge_msg(GA1, GB1, ea_a, ea_b, c1d, w2d, b2d, 2048)
    zeros = jnp.zeros((ZROWS, HID), jnp.float32)
    H2 = _scatter_h2(msg.reshape(EP, HID), dst_s, zeros)

    U2, V2 = _node_stage2(H2[0::2], H2[1::2], xe, xo, s, ss,
                          r1(bn_gamma), r1(bn_beta), A2, B2w, r1(eg_b1), 1000)
    GA2, GB2 = _gather_sum(U2.reshape(NN, CH), V2.reshape(NN, CH),
                           src_g, dst_g)
    outp = _edge_score(GA2, GB2, ea_a, ea_b, c2d, wd, bd, 2048)
    out = outp.reshape(EP)[:EDGES]
    return (out, jnp.zeros((NN, 3), jnp.float32))


# speedup vs baseline: 2.9004x; 1.3709x over previous
import functools

import jax
import jax.numpy as jnp
from jax import lax
from jax.experimental import pallas as pl
from jax.experimental.pallas import tpu as pltpu
from jax.experimental.pallas import tpu_sc as plsc

NN = 100000
EDGES = 1600000
HID = 32
CH = 64
EP = 1605632
ROWS128 = EP // 128
NW = 32
PER_W = EP // NW
CHUNK = 1024
G_ITERS = PER_W // CHUNK
PER_TEC_S = EP // 16
S_ITERS = PER_TEC_S // CHUNK
HALF = NN // 2
SPH_ROWS = 50048
ZROWS = SPH_ROWS // 16
SCHUNK = 256
SS_ITERS = PER_TEC_S // SCHUNK
BN_EPS = 1e-5


def _mesh():
    return plsc.VectorSubcoreMesh(core_axis_name="c", subcore_axis_name="s")


def _stats_body(x_ref, s_ref, ss_ref):
    i = pl.program_id(0)

    @pl.when(i == 0)
    def _():
        s_ref[...] = jnp.zeros_like(s_ref)
        ss_ref[...] = jnp.zeros_like(ss_ref)

    xb = x_ref[...]
    s_ref[...] += jnp.sum(xb, axis=0, keepdims=True)
    ss_ref[...] += jnp.sum(xb * xb, axis=0, keepdims=True)


def _node_stats(x, bn):
    return pl.pallas_call(
        _stats_body,
        grid=(NN // bn,),
        in_specs=[pl.BlockSpec((bn, 3), lambda i: (i, 0))],
        out_specs=[pl.BlockSpec((1, 3), lambda i: (0, 0)),
                   pl.BlockSpec((1, 3), lambda i: (0, 0))],
        out_shape=[jax.ShapeDtypeStruct((1, 3), jnp.float32),
                   jax.ShapeDtypeStruct((1, 3), jnp.float32)],
    )(x)


def _bn_scale_shift(s_ref, ss_ref, g_ref, b_ref):
    mean = s_ref[...] / NN
    var = ss_ref[...] / NN - mean * mean
    scale = g_ref[...] * lax.rsqrt(var + BN_EPS)
    shift = b_ref[...] - mean * scale
    return scale, shift


def _node1_body(xe_ref, xo_ref, s_ref, ss_ref, g_ref, b_ref, w1_ref, b1_ref,
                w2_ref, b2_ref, pw_ref, qw_ref, cb1_ref, p_ref, q_ref):
    scale, shift = _bn_scale_shift(s_ref, ss_ref, g_ref, b_ref)
    dot = functools.partial(jnp.dot, preferred_element_type=jnp.float32)
    pw = pw_ref[...]
    qw = qw_ref[...]

    def half(x_ref):
        X = x_ref[...] * scale + shift
        T = jnp.maximum(dot(X, w1_ref[...]) + b1_ref[...], 0.0)
        H = jnp.tanh(dot(T, w2_ref[...]) + b2_ref[...])
        P = dot(H, pw[:HID]) + dot(X, pw[HID:]) + cb1_ref[...]
        Q = dot(H, qw[:HID]) + dot(X, qw[HID:])
        return P, Q

    Pe, Qe = half(xe_ref)
    Po, Qo = half(xo_ref)
    p_ref[...] = jnp.concatenate([Pe, Po], axis=1)
    q_ref[...] = jnp.concatenate([Qe, Qo], axis=1)


def _node_stage1(xe, xo, s, ss, gamma, beta, w1, b1, w2, b2, pw, qw, cb1, bn2):
    full2 = lambda a: pl.BlockSpec(a.shape, lambda i: (0, 0))
    return pl.pallas_call(
        _node1_body,
        grid=(NN // 2 // bn2,),
        in_specs=[pl.BlockSpec((bn2, 3), lambda i: (i, 0)),
                  pl.BlockSpec((bn2, 3), lambda i: (i, 0)),
                  full2(s), full2(ss), full2(gamma), full2(beta),
                  full2(w1), full2(b1), full2(w2), full2(b2),
                  full2(pw), full2(qw), full2(cb1)],
        out_specs=[pl.BlockSpec((bn2, 2 * CH), lambda i: (i, 0)),
                   pl.BlockSpec((bn2, 2 * CH), lambda i: (i, 0))],
        out_shape=[jax.ShapeDtypeStruct((NN // 2, 2 * CH), jnp.float32),
                   jax.ShapeDtypeStruct((NN // 2, 2 * CH), jnp.float32)],
    )(xe, xo, s, ss, gamma, beta, w1, b1, w2, b2, pw, qw, cb1)


def _node2_body(h2e_ref, h2o_ref, xe_ref, xo_ref, s_ref, ss_ref, g_ref, b_ref,
                aw_ref, bw_ref, eb1_ref, u_ref, v_ref):
    scale, shift = _bn_scale_shift(s_ref, ss_ref, g_ref, b_ref)
    dot = functools.partial(jnp.dot, preferred_element_type=jnp.float32)
    aw = aw_ref[...]
    bw = bw_ref[...]

    def half(h2_ref, x_ref):
        X = x_ref[...] * scale + shift
        h2 = h2_ref[...]
        U = dot(h2, aw[:HID]) + dot(X, aw[HID:])
        V = dot(h2, bw[:HID]) + dot(X, bw[HID:]) + eb1_ref[...]
        return U, V

    Ue, Ve = half(h2e_ref, xe_ref)
    Uo, Vo = half(h2o_ref, xo_ref)
    u_ref[...] = jnp.concatenate([Ue, Uo], axis=1)
    v_ref[...] = jnp.concatenate([Ve, Vo], axis=1)


def _node_stage2(h2e, h2o, xe, xo, s, ss, gamma, beta, aw, bw, eb1, bn2):
    full2 = lambda a: pl.BlockSpec(a.shape, lambda i: (0, 0))
    return pl.pallas_call(
        _node2_body,
        grid=(NN // 2 // bn2,),
        in_specs=[pl.BlockSpec((bn2, HID), lambda i: (i, 0)),
                  pl.BlockSpec((bn2, HID), lambda i: (i, 0)),
                  pl.BlockSpec((bn2, 3), lambda i: (i, 0)),
                  pl.BlockSpec((bn2, 3), lambda i: (i, 0)),
                  full2(s), full2(ss), full2(gamma), full2(beta),
                  full2(aw), full2(bw), full2(eb1)],
        out_specs=[pl.BlockSpec((bn2, 2 * CH), lambda i: (i, 0)),
                   pl.BlockSpec((bn2, 2 * CH), lambda i: (i, 0))],
        out_shape=[jax.ShapeDtypeStruct((NN // 2, 2 * CH), jnp.float32),
                   jax.ShapeDtypeStruct((NN // 2, 2 * CH), jnp.float32)],
    )(h2e, h2o, xe, xo, s, ss, gamma, beta, aw, bw, eb1)


def _gather_sum(tabA, tabB, iA2d, iB2d):
    @functools.partial(
        pl.kernel,
        out_type=[jax.ShapeDtypeStruct((EP // 4, 2 * CH), jnp.float32),
                  jax.ShapeDtypeStruct((EP // 4, 2 * CH), jnp.float32)],
        mesh=_mesh(),
        scratch_types=[pltpu.VMEM((8, 128), jnp.int32),
                       pltpu.VMEM((8, 128), jnp.int32),
                       pltpu.VMEM((2, 128, CH), jnp.float32),
                       pltpu.VMEM((2, 128, CH), jnp.float32),
                       pltpu.VMEM((CHUNK // 4, 2 * CH), jnp.float32),
                       pltpu.VMEM((CHUNK // 4, 2 * CH), jnp.float32),
                       pltpu.SemaphoreType.DMA((2,))],
        compiler_params=pltpu.CompilerParams(use_tc_tiling_on_sc=False),
    )
    def k(tA, tB, iA, iB, outA, outB, ia_v, ib_v, abuf, bbuf, oa, ob, sem):
        wid = lax.axis_index("s") * 2 + lax.axis_index("c")
        row0 = wid * (PER_W // 128)

        def chunk(i):
            pltpu.sync_copy(iA.at[pl.ds(row0 + i * 8, 8)], ia_v)
            pltpu.sync_copy(iB.at[pl.ds(row0 + i * 8, 8)], ib_v)
            cps = [pltpu.async_copy(tA.at[ia_v.at[0]], abuf.at[0], sem.at[0]),
                   pltpu.async_copy(tB.at[ib_v.at[0]], bbuf.at[0], sem.at[0])]
            for j in range(8):
                b = j % 2
                nb = (j + 1) % 2
                if j < 7:
                    cps += [pltpu.async_copy(tA.at[ia_v.at[j + 1]],
                                             abuf.at[nb], sem.at[nb]),
                            pltpu.async_copy(tB.at[ib_v.at[j + 1]],
                                             bbuf.at[nb], sem.at[nb])]
                cps.pop(0).wait()
                cps.pop(0).wait()

                def quad(r4, j=j, b=b):
                    row = j * 32 + r4
                    for t in range(4):
                        dst = oa if t < 2 else ob
                        off = (t % 2) * CH
                        for c4 in range(CH // 16):
                            sl = pl.ds(c4 * 16, 16)
                            dst[row, pl.ds(off + c4 * 16, 16)] = (
                                abuf[b, 4 * r4 + t, sl]
                                + bbuf[b, 4 * r4 + t, sl])

                pl.loop(0, 32)(quad)
            base = wid * (PER_W // 4) + i * (CHUNK // 4)
            pltpu.sync_copy(oa, outA.at[pl.ds(base, CHUNK // 4)])
            pltpu.sync_copy(ob, outB.at[pl.ds(base, CHUNK // 4)])

        pl.loop(0, G_ITERS)(chunk)

    return k(tabA, tabB, iA2d, iB2d)


def _scatter_h2(msg, dsts2d, zeros):
    @functools.partial(
        pl.kernel,
        out_type=jax.ShapeDtypeStruct((NN, HID), jnp.float32),
        mesh=_mesh(),
        scratch_types=[pltpu.VMEM((2, 128), jnp.int32),
                       pltpu.VMEM((2, 128), jnp.int32),
                       pltpu.VMEM((SCHUNK, HID), jnp.float32),
                       pltpu.VMEM_SHARED((SPH_ROWS, HID), jnp.float32),
                       pltpu.SemaphoreType.DMA],
        compiler_params=pltpu.CompilerParams(use_tc_tiling_on_sc=False),
    )
    def k(msg_hbm, dsts_hbm, zeros_hbm, out, idx_v, lidx_v, mbuf, acc, sem):
        c = lax.axis_index("c")
        t = lax.axis_index("s")
        base = c * HALF
        pltpu.sync_copy(zeros_hbm, acc.at[pl.ds(t * ZROWS, ZROWS)])
        plsc.subcore_barrier()
        row_base = t * (PER_TEC_S // 128)
        e_base = t * PER_TEC_S

        def chunk(i):
            pltpu.sync_copy(dsts_hbm.at[pl.ds(row_base + i * 2, 2)], idx_v)
            pltpu.sync_copy(msg_hbm.at[pl.ds(e_base + i * SCHUNK, SCHUNK)],
                            mbuf)
            for j in range(2):
                for kk in range(8):
                    sl = pl.ds(kk * 16, 16)
                    v = idx_v[j, sl] - base
                    m = (v >= 0) & (v < HALF)
                    lidx_v[j, sl] = jnp.where(m, v, HALF)
            cps = [pltpu.async_copy(mbuf.at[pl.ds(j * 128, 128)],
                                    acc.at[lidx_v.at[j]], sem, add=True)
                   for j in range(2)]
            for cp in cps:
                cp.wait()

        pl.loop(0, SS_ITERS)(chunk)
        plsc.subcore_barrier()

        @pl.when(t == 0)
        def _():
            pltpu.sync_copy(acc.at[pl.ds(0, HALF)],
                            out.at[pl.ds(base, HALF)])

        plsc.subcore_barrier()

    return k(msg, dsts2d, zeros)


def _ea_dot(ea_ref, c_ref):
    return lax.dot_general(ea_ref[...], c_ref[...], (((0,), (0,)), ((), ())),
                           preferred_element_type=jnp.float32)


def _b2_body(ga_ref, gb_ref, e0_ref, e1_ref, e2_ref, e3_ref,
             c1_ref, w2d_ref, b2d_ref, o_ref):
    dot = functools.partial(jnp.dot, preferred_element_type=jnp.float32)
    w2d = w2d_ref[...]
    b2d = b2d_ref[...]
    ea01 = jnp.concatenate([_ea_dot(e0_ref, c1_ref),
                            _ea_dot(e1_ref, c1_ref)], axis=1)
    ea23 = jnp.concatenate([_ea_dot(e2_ref, c1_ref),
                            _ea_dot(e3_ref, c1_ref)], axis=1)
    ha = jnp.maximum(ga_ref[...] + ea01, 0.0)
    hb = jnp.maximum(gb_ref[...] + ea23, 0.0)
    msga = jnp.tanh(dot(ha, w2d) + b2d)
    msgb = jnp.tanh(dot(hb, w2d) + b2d)
    o_ref[...] = jnp.concatenate([msga, msgb], axis=1)


def _ea_specs(be4):
    nq = (EP // 4) // be4
    return [pl.BlockSpec((4, be4), lambda i, k=k: (0, k * nq + i))
            for k in range(4)]


def _edge_msg(GA, GB, eaT, c1, w2d, b2d, be):
    full2 = lambda a: pl.BlockSpec(a.shape, lambda i: (0, 0))
    be4 = be // 4
    return pl.pallas_call(
        _b2_body,
        grid=(EP // be,),
        in_specs=[pl.BlockSpec((be4, 2 * CH), lambda i: (i, 0)),
                  pl.BlockSpec((be4, 2 * CH), lambda i: (i, 0))]
        + _ea_specs(be4)
        + [full2(c1), full2(w2d), full2(b2d)],
        out_specs=[pl.BlockSpec((be4, 4 * HID), lambda i: (i, 0))],
        out_shape=[jax.ShapeDtypeStruct((EP // 4, 4 * HID), jnp.float32)],
    )(GA, GB, eaT, eaT, eaT, eaT, c1, w2d, b2d)[0]


def _b3_body(ga_ref, gb_ref, e0_ref, e1_ref, e2_ref, e3_ref,
             c2_ref, wd_ref, bd_ref, o_ref):
    dot = functools.partial(jnp.dot, preferred_element_type=jnp.float32)
    wd = wd_ref[...]
    bd = bd_ref[...]
    ea01 = jnp.concatenate([_ea_dot(e0_ref, c2_ref),
                            _ea_dot(e1_ref, c2_ref)], axis=1)
    ea23 = jnp.concatenate([_ea_dot(e2_ref, c2_ref),
                            _ea_dot(e3_ref, c2_ref)], axis=1)
    ha = jnp.maximum(ga_ref[...] + ea01, 0.0)
    hb = jnp.maximum(gb_ref[...] + ea23, 0.0)
    sa = jax.nn.sigmoid(dot(ha, wd) + bd)
    sb = jax.nn.sigmoid(dot(hb, wd) + bd)
    o_ref[...] = jnp.concatenate([sa, sb], axis=1)


def _edge_score(GA, GB, eaT, c2, wd, bd, be):
    full2 = lambda a: pl.BlockSpec(a.shape, lambda i: (0, 0))
    be4 = be // 4
    return pl.pallas_call(
        _b3_body,
        grid=(EP // be,),
        in_specs=[pl.BlockSpec((be4, 2 * CH), lambda i: (i, 0)),
                  pl.BlockSpec((be4, 2 * CH), lambda i: (i, 0))]
        + _ea_specs(be4)
        + [full2(c2), full2(wd), full2(bd)],
        out_specs=[pl.BlockSpec((be4, 4), lambda i: (i, 0))],
        out_shape=[jax.ShapeDtypeStruct((EP // 4, 4), jnp.float32)],
    )(GA, GB, eaT, eaT, eaT, eaT, c2, wd, bd)[0]


def _blockdiag2(w):
    z = jnp.zeros_like(w)
    return jnp.concatenate(
        [jnp.concatenate([w, z], axis=1), jnp.concatenate([z, w], axis=1)],
        axis=0)


def kernel(x, edge_index, edge_attr, bn_gamma, bn_beta,
           in_W1, in_b1, in_W2, in_b2,
           conv_W1, conv_b1, conv_W2, conv_b2,
           eg_W1, eg_b1, eg_W2, eg_b2):
    src = edge_index[0]
    dst = edge_index[1]
    pad = EP - EDGES
    Q4 = EP // 4
    perm = lambda a: a.reshape(4, Q4).T.reshape(ROWS128, 128)
    src_g = perm(jnp.pad(src, (0, pad)))
    dst_g = perm(jnp.pad(dst, (0, pad)))
    dst_s = perm(jnp.pad(dst, (0, pad), constant_values=NN))
    eaT = jnp.pad(edge_attr.T, ((0, 0), (0, pad)))

    A1 = conv_W1[:HID + 3]
    B1 = conv_W1[HID + 3:2 * (HID + 3)]
    C1 = conv_W1[2 * (HID + 3):]
    pw = A1 - B1
    qw = B1
    A2 = eg_W1[:HID + 3]
    B2w = eg_W1[HID + 3:2 * (HID + 3)]
    C2 = eg_W1[2 * (HID + 3):]

    w2d = _blockdiag2(conv_W2)
    b2d = jnp.tile(conv_b2, 2).reshape(1, 2 * HID)
    wd = _blockdiag2(eg_W2)
    bd = jnp.tile(eg_b2, 2).reshape(1, 2)

    r1 = lambda a: a.reshape(1, -1)
    xe = x[0::2]
    xo = x[1::2]
    s, ss = _node_stats(x, 2000)
    P2, Q2 = _node_stage1(xe, xo, s, ss, r1(bn_gamma), r1(bn_beta),
                          in_W1, r1(in_b1), in_W2, r1(in_b2),
                          pw, qw, r1(conv_b1), 1000)

    GA1, GB1 = _gather_sum(P2.reshape(NN, CH), Q2.reshape(NN, CH),
                           dst_g, src_g)
    msg = _edge_msg(GA1, GB1, eaT, C1, w2d, b2d, 2048)
    zeros = jnp.zeros((ZROWS, HID), jnp.float32)
    H2 = _scatter_h2(msg.reshape(EP, HID), dst_s, zeros)

    U2, V2 = _node_stage2(H2[0::2], H2[1::2], xe, xo, s, ss,
                          r1(bn_gamma), r1(bn_beta), A2, B2w, r1(eg_b1), 1000)
    GA2, GB2 = _gather_sum(U2.reshape(NN, CH), V2.reshape(NN, CH),
                           src_g, dst_g)
    outp = _edge_score(GA2, GB2, eaT, C2, wd, bd, 2048)
    out = outp.T.reshape(EP)[:EDGES]
    return (out, jnp.zeros((NN, 3), jnp.float32))
